# baseline (speedup 1.0000x reference)
import functools

import jax
import jax.numpy as jnp
from jax import lax
from jax.experimental import pallas as pl
from jax.experimental.pallas import tpu as pltpu
from jax.experimental.pallas import tpu_sc as plsc

N = 10000
D = 128
H = 16
C = 10

NC = 2
NS = 16
NW = NC * NS
BATCH = 128
NPAD = 10240
CHUNK = NPAD // NS

_MESH = plsc.VectorSubcoreMesh(core_axis_name="c", subcore_axis_name="s")
_SC_PARAMS = pltpu.CompilerParams(
    use_tc_tiling_on_sc=False, needs_layout_passes=False
)


def _num_batches(e):
    nb = -(-e // (NW * BATCH))
    if nb % 2 == 0:
        nb += 1
    return nb


def _rsqrt16(v):
    i = plsc.bitcast(v, jnp.int32)
    i = jnp.int32(0x5F3759DF) - (i >> 1)
    y = plsc.bitcast(i, jnp.float32)
    for _ in range(3):
        y = y * (1.5 - 0.5 * v * y * y)
    return y


def _zero_acc_slice(zbuf, acc, s):
    def zrow(i, _):
        zbuf[i, :] = jnp.zeros((16,), jnp.float32)
        return 0

    lax.fori_loop(0, CHUNK, zrow, 0)
    pltpu.sync_copy(zbuf, acc.at[pl.ds(s * CHUNK, CHUNK)])


def _edge_pass(src_v, dst_v, g_sh, acc, rows0, rows1, sem0, sem1, nb):
    pltpu.async_copy(g_sh.at[src_v.at[0]], rows0, sem0)

    def pair(p, _):
        j0 = p * 2
        pltpu.async_copy(g_sh.at[src_v.at[j0 + 1]], rows1, sem1)
        pltpu.make_async_copy(g_sh.at[src_v.at[j0]], rows0, sem0).wait()
        pltpu.sync_copy(rows0, acc.at[dst_v.at[j0]], add=True)
        pltpu.async_copy(g_sh.at[src_v.at[j0 + 2]], rows0, sem0)
        pltpu.make_async_copy(g_sh.at[src_v.at[j0 + 1]], rows1, sem1).wait()
        pltpu.sync_copy(rows1, acc.at[dst_v.at[j0 + 1]], add=True)
        return 0

    lax.fori_loop(0, (nb - 1) // 2, pair, 0)
    pltpu.make_async_copy(g_sh.at[src_v.at[nb - 1]], rows0, sem0).wait()
    pltpu.sync_copy(rows0, acc.at[dst_v.at[nb - 1]], add=True)


def _make_sc_layer1(nb):
    @functools.partial(
        pl.kernel,
        out_type=(
            jax.ShapeDtypeStruct((NC, NPAD, 16), jnp.float32),
            jax.ShapeDtypeStruct((NPAD, 16), jnp.float32),
            jax.ShapeDtypeStruct((NPAD, 16), jnp.float32),
        ),
        mesh=_MESH,
        compiler_params=_SC_PARAMS,
        scratch_types=[
            pltpu.VMEM((nb, BATCH), jnp.int32),
            pltpu.VMEM((nb, BATCH), jnp.int32),
            pltpu.VMEM((nb, BATCH), jnp.int32),
            pltpu.VMEM((BATCH,), jnp.float32),
            pltpu.VMEM((CHUNK,), jnp.float32),
            pltpu.VMEM((CHUNK,), jnp.float32),
            pltpu.VMEM((CHUNK, 16), jnp.float32),
            pltpu.VMEM((CHUNK, 16), jnp.float32),
            pltpu.VMEM((CHUNK, 16), jnp.float32),
            pltpu.VMEM((CHUNK, 16), jnp.float32),
            pltpu.VMEM((BATCH, 16), jnp.float32),
            pltpu.VMEM((BATCH, 16), jnp.float32),
            pltpu.VMEM_SHARED((NPAD,), jnp.float32),
            pltpu.VMEM_SHARED((NPAD, 16), jnp.float32),
            pltpu.VMEM_SHARED((NPAD, 16), jnp.float32),
            pltpu.SemaphoreType.DMA,
            pltpu.SemaphoreType.DMA,
        ],
    )
    def sc_layer1(
        src_hbm, dst_hbm, h_hbm, out_hbm, g_hbm, dinv_hbm,
        src_v, dst_v, dst2_v, ones_v, degc, dinvc, zbuf, hc, gc, dc,
        rows0, rows1, deg_sh, g_sh, acc, sem0, sem1,
    ):
        c = lax.axis_index("c")
        s = lax.axis_index("s")
        wid = c * NS + s
        wid2 = (1 - c) * NS + s

        for k in range(CHUNK // 16):
            degc[pl.ds(k * 16, 16)] = jnp.zeros((16,), jnp.float32)
        for k in range(BATCH // 16):
            ones_v[pl.ds(k * 16, 16)] = jnp.ones((16,), jnp.float32)
        pltpu.sync_copy(degc, deg_sh.at[pl.ds(s * CHUNK, CHUNK)])
        _zero_acc_slice(zbuf, acc, s)
        pltpu.sync_copy(src_hbm.at[wid], src_v)
        pltpu.sync_copy(dst_hbm.at[wid], dst_v)
        pltpu.sync_copy(dst_hbm.at[wid2], dst2_v)
        pltpu.sync_copy(
            h_hbm.at[pl.ds(s * CHUNK, CHUNK), pl.ds(0, 16)], hc
        )
        plsc.subcore_barrier()

        def cnt(j, _):
            pltpu.sync_copy(ones_v, deg_sh.at[dst_v.at[j]], add=True)
            pltpu.sync_copy(ones_v, deg_sh.at[dst2_v.at[j]], add=True)
            return 0

        lax.fori_loop(0, nb, cnt, 0)
        plsc.subcore_barrier()

        pltpu.sync_copy(deg_sh.at[pl.ds(s * CHUNK, CHUNK)], degc)
        for k in range(CHUNK // 16):
            v = degc[pl.ds(k * 16, 16)] + 1.0
            dinvc[pl.ds(k * 16, 16)] = _rsqrt16(v)

        def brow(r, _):
            dsp = plsc.load_gather(dinvc, [jnp.full((16,), r, jnp.int32)])
            gc[r, :] = hc[r, :] * dsp
            dc[r, :] = dsp
            return 0

        lax.fori_loop(0, CHUNK, brow, 0)
        pltpu.sync_copy(gc, g_sh.at[pl.ds(s * CHUNK, CHUNK)])

        @pl.when(c == 0)
        def _():
            pltpu.sync_copy(gc, g_hbm.at[pl.ds(s * CHUNK, CHUNK)])
            pltpu.sync_copy(dc, dinv_hbm.at[pl.ds(s * CHUNK, CHUNK)])

        plsc.subcore_barrier()

        _edge_pass(src_v, dst_v, g_sh, acc, rows0, rows1, sem0, sem1, nb)
        plsc.subcore_barrier()
        pltpu.sync_copy(
            acc.at[pl.ds(s * CHUNK, CHUNK)],
            out_hbm.at[c, pl.ds(s * CHUNK, CHUNK)],
        )

    return sc_layer1


def _make_sc_layer2(nb):
    @functools.partial(
        pl.kernel,
        out_type=jax.ShapeDtypeStruct((NC, NPAD, 16), jnp.float32),
        mesh=_MESH,
        compiler_params=_SC_PARAMS,
        scratch_types=[
            pltpu.VMEM((nb, BATCH), jnp.int32),
            pltpu.VMEM((nb, BATCH), jnp.int32),
            pltpu.VMEM((BATCH, 16), jnp.float32),
            pltpu.VMEM((BATCH, 16), jnp.float32),
            pltpu.VMEM((CHUNK, 16), jnp.float32),
            pltpu.VMEM_SHARED((NPAD, 16), jnp.float32),
            pltpu.VMEM_SHARED((NPAD, 16), jnp.float32),
            pltpu.SemaphoreType.DMA,
            pltpu.SemaphoreType.DMA,
        ],
    )
    def sc_layer2(
        src_hbm, dst_hbm, g_hbm, out_hbm,
        src_v, dst_v, rows0, rows1, zbuf, g_sh, acc, sem0, sem1,
    ):
        c = lax.axis_index("c")
        s = lax.axis_index("s")
        wid = c * NS + s

        _zero_acc_slice(zbuf, acc, s)
        pltpu.sync_copy(src_hbm.at[wid], src_v)
        pltpu.sync_copy(dst_hbm.at[wid], dst_v)
        pltpu.sync_copy(
            g_hbm.at[pl.ds(s * CHUNK, CHUNK)],
            g_sh.at[pl.ds(s * CHUNK, CHUNK)],
        )
        plsc.subcore_barrier()
        _edge_pass(src_v, dst_v, g_sh, acc, rows0, rows1, sem0, sem1, nb)
        plsc.subcore_barrier()
        pltpu.sync_copy(
            acc.at[pl.ds(s * CHUNK, CHUNK)],
            out_hbm.at[c, pl.ds(s * CHUNK, CHUNK)],
        )

    return sc_layer2


NF = NPAD * 16 // 128


def _tc1_body(x_ref, w_ref, h_ref):
    h_ref[...] = jnp.dot(
        x_ref[...], w_ref[...], preferred_element_type=jnp.float32, precision=lax.Precision.HIGHEST
    )


def _tc2_body(s_ref, g_ref, dinv_ref, b_ref, w_ref, o_ref):
    tot = s_ref[:NF] + s_ref[NF:] + g_ref[...]
    h2 = jnp.maximum(dinv_ref[...] * tot + b_ref[...], 0.0)
    o_ref[...] = (
        jnp.dot(h2, w_ref[...], preferred_element_type=jnp.float32, precision=lax.Precision.HIGHEST)
        * dinv_ref[...]
    )


def _tc3_body(s_ref, g_ref, dinv_ref, b_ref, ones_ref, o_ref):
    o = dinv_ref[...] * (s_ref[:NF] + s_ref[NF:] + g_ref[...]) + b_ref[...]
    col = lax.broadcasted_iota(jnp.int32, o.shape, 1) % 16
    valid = col < C
    m = jnp.max(jnp.where(valid, o, -jnp.inf), axis=1, keepdims=True)
    om = o - m
    e = jnp.where(valid, jnp.exp(om), 0.0)
    ssum = jnp.dot(e, ones_ref[...], preferred_element_type=jnp.float32, precision=lax.Precision.HIGHEST)
    o_ref[...] = om - jnp.log(ssum)


def _flat_spec(rows):
    return pl.BlockSpec((rows, 128), lambda: (0, 0))


_tc1 = pl.pallas_call(
    _tc1_body,
    in_specs=[_flat_spec(NPAD), _flat_spec(D)],
    out_specs=_flat_spec(NPAD),
    out_shape=jax.ShapeDtypeStruct((NPAD, 128), jnp.float32),
)

_tc2 = pl.pallas_call(
    _tc2_body,
    in_specs=[
        _flat_spec(2 * NF),
        _flat_spec(NF),
        _flat_spec(NF),
        pl.BlockSpec((1, 128), lambda: (0, 0)),
        _flat_spec(128),
    ],
    out_specs=_flat_spec(NF),
    out_shape=jax.ShapeDtypeStruct((NF, 128), jnp.float32),
)

_tc3 = pl.pallas_call(
    _tc3_body,
    in_specs=[
        _flat_spec(2 * NF),
        _flat_spec(NF),
        _flat_spec(NF),
        pl.BlockSpec((1, 128), lambda: (0, 0)),
        _flat_spec(128),
    ],
    out_specs=_flat_spec(NF),
    out_shape=jax.ShapeDtypeStruct((NF, 128), jnp.float32),
)


def kernel(x, edge_index, W1, b1, W2, b2):
    src = edge_index[0]
    dst = edge_index[1]
    e = src.shape[0]
    nb = _num_batches(e)
    epad = NW * nb * BATCH

    src_sl = jnp.concatenate(
        [src, jnp.zeros((epad - e,), jnp.int32)]
    ).reshape(NW, nb, BATCH)
    dst_sl = jnp.concatenate(
        [dst, jnp.full((epad - e,), N, jnp.int32)]
    ).reshape(NW, nb, BATCH)
    x_pad = jnp.pad(x, ((0, NPAD - N), (0, 0)))

    eye8 = jnp.eye(8, dtype=jnp.float32)
    w2p = jnp.pad(W2, ((0, 0), (0, 16 - C)))
    w2blk = jnp.kron(eye8, w2p)
    onesblk = jnp.kron(eye8, jnp.ones((16, 16), jnp.float32))
    b1t = jnp.tile(b1, 8).reshape(1, 128)
    b2t = jnp.tile(jnp.pad(b2, (0, 16 - C)), 8).reshape(1, 128)

    w1p = jnp.pad(W1, ((0, 0), (0, 128 - H)))
    y1 = _tc1(x_pad, w1p)
    s1, g1, dinv = _make_sc_layer1(nb)(src_sl, dst_sl, y1)

    s1f = s1.reshape(2 * NF, 128)
    g1f = g1.reshape(NF, 128)
    dinvf = dinv.reshape(NF, 128)
    g2f = _tc2(s1f, g1f, dinvf, b1t, w2blk)

    s2 = _make_sc_layer2(nb)(src_sl, dst_sl, g2f.reshape(NPAD, 16))
    of = _tc3(s2.reshape(2 * NF, 128), g2f, dinvf, b2t, onesblk)
    return of.reshape(NPAD, 16)[:N, :C]


# speedup vs baseline: 1.1197x; 1.1197x over previous
import functools

import jax
import jax.numpy as jnp
from jax import lax
from jax.experimental import pallas as pl
from jax.experimental.pallas import tpu as pltpu
from jax.experimental.pallas import tpu_sc as plsc

N = 10000
D = 128
H = 16
C = 10

NC = 2
NS = 16
NW = NC * NS
BATCH = 128
NPAD = 10240
CHUNK = NPAD // NS

_MESH = plsc.VectorSubcoreMesh(core_axis_name="c", subcore_axis_name="s")
_SC_PARAMS = pltpu.CompilerParams(
    use_tc_tiling_on_sc=False, needs_layout_passes=False
)


def _num_batches(e):
    nb = -(-e // (NW * BATCH))
    if nb % 2 == 0:
        nb += 1
    return nb


def _rsqrt16(v):
    i = plsc.bitcast(v, jnp.int32)
    i = jnp.int32(0x5F3759DF) - (i >> 1)
    y = plsc.bitcast(i, jnp.float32)
    for _ in range(3):
        y = y * (1.5 - 0.5 * v * y * y)
    return y


def _zero_acc_slice(zbuf, acc, s):
    def zrow(i, _):
        zbuf[i, :] = jnp.zeros((16,), jnp.float32)
        return 0

    lax.fori_loop(0, CHUNK, zrow, 0)
    pltpu.sync_copy(zbuf, acc.at[pl.ds(s * CHUNK, CHUNK)])


def _edge_pass(src_v, dst_v, g_sh, acc, bufs, gsems, ssems, nb):
    def fire_g(j, b):
        pltpu.async_copy(g_sh.at[src_v.at[j]], bufs[b], gsems[b])

    def wait_g(j, b):
        pltpu.make_async_copy(g_sh.at[src_v.at[j]], bufs[b], gsems[b]).wait()

    def fire_s(j, b):
        pltpu.async_copy(bufs[b], acc.at[dst_v.at[j]], ssems[b], add=True)

    def wait_s(j, b):
        pltpu.make_async_copy(
            bufs[b], acc.at[dst_v.at[j]], ssems[b]
        ).wait()

    def step(j, b, do_wait_s, do_fire_g):
        b2 = (b + 2) % 4
        if do_wait_s:
            wait_s(j - 2, b2)
        if do_fire_g:
            fire_g(j + 2, b2)
        wait_g(j, b)
        fire_s(j, b)

    fire_g(0, 0)
    fire_g(1, 1)
    step(0, 0, False, True)
    step(1, 1, False, True)

    def body(q, _):
        j0 = 2 + q * 4
        for r in range(4):
            step(j0 + r, (2 + r) % 4, True, True)
        return 0

    nq = (nb - 6) // 4
    lax.fori_loop(0, nq, body, 0)
    for j in range(2 + 4 * nq, nb - 2):
        step(j, j % 4, True, True)
    for j in (nb - 2, nb - 1):
        step(j, j % 4, True, False)
    wait_s(nb - 2, (nb - 2) % 4)
    wait_s(nb - 1, (nb - 1) % 4)


def _make_sc_layer1(nb):
    @functools.partial(
        pl.kernel,
        out_type=(
            jax.ShapeDtypeStruct((NC, NPAD, 16), jnp.float32),
            jax.ShapeDtypeStruct((NPAD, 16), jnp.float32),
            jax.ShapeDtypeStruct((NPAD, 16), jnp.float32),
        ),
        mesh=_MESH,
        compiler_params=_SC_PARAMS,
        scratch_types=[
            pltpu.VMEM((nb, BATCH), jnp.int32),
            pltpu.VMEM((nb, BATCH), jnp.int32),
            pltpu.VMEM((nb, BATCH), jnp.int32),
            pltpu.VMEM((BATCH,), jnp.float32),
            pltpu.VMEM((CHUNK,), jnp.float32),
            pltpu.VMEM((CHUNK,), jnp.float32),
            pltpu.VMEM((CHUNK, 16), jnp.float32),
            pltpu.VMEM((CHUNK, 16), jnp.float32),
            pltpu.VMEM((CHUNK, 16), jnp.float32),
            pltpu.VMEM((CHUNK, 16), jnp.float32),
            [pltpu.VMEM((BATCH, 16), jnp.float32) for _ in range(4)],
            pltpu.VMEM_SHARED((NPAD,), jnp.float32),
            pltpu.VMEM_SHARED((NPAD, 16), jnp.float32),
            pltpu.VMEM_SHARED((NPAD, 16), jnp.float32),
            [pltpu.SemaphoreType.DMA for _ in range(4)],
            [pltpu.SemaphoreType.DMA for _ in range(4)],
            pltpu.SemaphoreType.DMA,
        ],
    )
    def sc_layer1(
        src_hbm, dst_hbm, h_hbm, out_hbm, g_hbm, dinv_hbm,
        src_v, dst_v, dst2_v, ones_v, degc, dinvc, zbuf, hc, gc, dc,
        bufs, deg_sh, g_sh, acc, gsems, ssems, semc,
    ):
        c = lax.axis_index("c")
        s = lax.axis_index("s")
        wid = c * NS + s
        wid2 = (1 - c) * NS + s

        for k in range(CHUNK // 16):
            degc[pl.ds(k * 16, 16)] = jnp.zeros((16,), jnp.float32)
        for k in range(BATCH // 16):
            ones_v[pl.ds(k * 16, 16)] = jnp.ones((16,), jnp.float32)
        pltpu.sync_copy(degc, deg_sh.at[pl.ds(s * CHUNK, CHUNK)])
        _zero_acc_slice(zbuf, acc, s)
        pltpu.sync_copy(src_hbm.at[wid], src_v)
        pltpu.sync_copy(dst_hbm.at[wid], dst_v)
        pltpu.sync_copy(dst_hbm.at[wid2], dst2_v)
        pltpu.sync_copy(
            h_hbm.at[pl.ds(s * CHUNK, CHUNK), pl.ds(0, 16)], hc
        )
        plsc.subcore_barrier()

        def count_slab(dvv):
            def fire(j):
                pltpu.async_copy(ones_v, deg_sh.at[dvv.at[j]], semc, add=True)

            def drain():
                pltpu.make_async_copy(
                    ones_v, deg_sh.at[dvv.at[0]], semc
                ).wait()

            for j in range(4):
                fire(j)

            def cnt(j, _):
                drain()
                fire(j)
                return 0

            lax.fori_loop(4, nb, cnt, 0)
            for _ in range(4):
                drain()

        count_slab(dst_v)
        count_slab(dst2_v)
        plsc.subcore_barrier()

        pltpu.sync_copy(deg_sh.at[pl.ds(s * CHUNK, CHUNK)], degc)
        for k in range(CHUNK // 16):
            v = degc[pl.ds(k * 16, 16)] + 1.0
            dinvc[pl.ds(k * 16, 16)] = _rsqrt16(v)

        def brow(r, _):
            dsp = plsc.load_gather(dinvc, [jnp.full((16,), r, jnp.int32)])
            gc[r, :] = hc[r, :] * dsp
            dc[r, :] = dsp
            return 0

        lax.fori_loop(0, CHUNK, brow, 0)
        pltpu.sync_copy(gc, g_sh.at[pl.ds(s * CHUNK, CHUNK)])

        @pl.when(c == 0)
        def _():
            pltpu.sync_copy(gc, g_hbm.at[pl.ds(s * CHUNK, CHUNK)])
            pltpu.sync_copy(dc, dinv_hbm.at[pl.ds(s * CHUNK, CHUNK)])

        plsc.subcore_barrier()

        _edge_pass(src_v, dst_v, g_sh, acc, bufs, gsems, ssems, nb)
        plsc.subcore_barrier()
        pltpu.sync_copy(
            acc.at[pl.ds(s * CHUNK, CHUNK)],
            out_hbm.at[c, pl.ds(s * CHUNK, CHUNK)],
        )

    return sc_layer1


def _make_sc_layer2(nb):
    @functools.partial(
        pl.kernel,
        out_type=jax.ShapeDtypeStruct((NC, NPAD, 16), jnp.float32),
        mesh=_MESH,
        compiler_params=_SC_PARAMS,
        scratch_types=[
            pltpu.VMEM((nb, BATCH), jnp.int32),
            pltpu.VMEM((nb, BATCH), jnp.int32),
            [pltpu.VMEM((BATCH, 16), jnp.float32) for _ in range(4)],
            pltpu.VMEM((CHUNK, 16), jnp.float32),
            pltpu.VMEM_SHARED((NPAD, 16), jnp.float32),
            pltpu.VMEM_SHARED((NPAD, 16), jnp.float32),
            [pltpu.SemaphoreType.DMA for _ in range(4)],
            [pltpu.SemaphoreType.DMA for _ in range(4)],
        ],
    )
    def sc_layer2(
        src_hbm, dst_hbm, g_hbm, out_hbm,
        src_v, dst_v, bufs, zbuf, g_sh, acc, gsems, ssems,
    ):
        c = lax.axis_index("c")
        s = lax.axis_index("s")
        wid = c * NS + s

        _zero_acc_slice(zbuf, acc, s)
        pltpu.sync_copy(src_hbm.at[wid], src_v)
        pltpu.sync_copy(dst_hbm.at[wid], dst_v)
        pltpu.sync_copy(
            g_hbm.at[pl.ds(s * CHUNK, CHUNK)],
            g_sh.at[pl.ds(s * CHUNK, CHUNK)],
        )
        plsc.subcore_barrier()
        _edge_pass(src_v, dst_v, g_sh, acc, bufs, gsems, ssems, nb)
        plsc.subcore_barrier()
        pltpu.sync_copy(
            acc.at[pl.ds(s * CHUNK, CHUNK)],
            out_hbm.at[c, pl.ds(s * CHUNK, CHUNK)],
        )

    return sc_layer2


NF = NPAD * 16 // 128


def _tc1_body(x_ref, w_ref, h_ref):
    h = jnp.dot(
        x_ref[...], w_ref[...],
        preferred_element_type=jnp.float32,
        precision=lax.Precision.HIGHEST,
    )
    h_ref[pl.ds(0, N), :] = h
    h_ref[pl.ds(N, NPAD - N), :] = jnp.zeros((NPAD - N, 128), jnp.float32)


def _tc2_body(s_ref, g_ref, dinv_ref, b_ref, w_ref, o_ref):
    tot = s_ref[:NF] + s_ref[NF:] + g_ref[...]
    h2 = jnp.maximum(dinv_ref[...] * tot + b_ref[...], 0.0)
    o_ref[...] = (
        jnp.dot(h2, w_ref[...], preferred_element_type=jnp.float32, precision=lax.Precision.HIGHEST)
        * dinv_ref[...]
    )


def _tc3_body(s_ref, g_ref, dinv_ref, b_ref, ones_ref, o_ref):
    o = dinv_ref[...] * (s_ref[:NF] + s_ref[NF:] + g_ref[...]) + b_ref[...]
    col = lax.broadcasted_iota(jnp.int32, o.shape, 1) % 16
    valid = col < C
    m = jnp.max(jnp.where(valid, o, -jnp.inf), axis=1, keepdims=True)
    om = o - m
    e = jnp.where(valid, jnp.exp(om), 0.0)
    ssum = jnp.dot(e, ones_ref[...], preferred_element_type=jnp.float32, precision=lax.Precision.HIGHEST)
    o_ref[...] = om - jnp.log(ssum)


def _flat_spec(rows):
    return pl.BlockSpec((rows, 128), lambda: (0, 0))


_tc1 = pl.pallas_call(
    _tc1_body,
    in_specs=[_flat_spec(N), _flat_spec(D)],
    out_specs=_flat_spec(NPAD),
    out_shape=jax.ShapeDtypeStruct((NPAD, 128), jnp.float32),
)

_tc2 = pl.pallas_call(
    _tc2_body,
    in_specs=[
        _flat_spec(2 * NF),
        _flat_spec(NF),
        _flat_spec(NF),
        pl.BlockSpec((1, 128), lambda: (0, 0)),
        _flat_spec(128),
    ],
    out_specs=_flat_spec(NF),
    out_shape=jax.ShapeDtypeStruct((NF, 128), jnp.float32),
)

_tc3 = pl.pallas_call(
    _tc3_body,
    in_specs=[
        _flat_spec(2 * NF),
        _flat_spec(NF),
        _flat_spec(NF),
        pl.BlockSpec((1, 128), lambda: (0, 0)),
        _flat_spec(128),
    ],
    out_specs=_flat_spec(NF),
    out_shape=jax.ShapeDtypeStruct((NF, 128), jnp.float32),
)


def kernel(x, edge_index, W1, b1, W2, b2):
    src = edge_index[0]
    dst = edge_index[1]
    e = src.shape[0]
    nb = _num_batches(e)
    epad = NW * nb * BATCH

    src_sl = jnp.concatenate(
        [src, jnp.zeros((epad - e,), jnp.int32)]
    ).reshape(NW, nb, BATCH)
    dst_sl = jnp.concatenate(
        [dst, jnp.full((epad - e,), N, jnp.int32)]
    ).reshape(NW, nb, BATCH)
    eye8 = jnp.eye(8, dtype=jnp.float32)
    w2p = jnp.pad(W2, ((0, 0), (0, 16 - C)))
    w2blk = jnp.kron(eye8, w2p)
    onesblk = jnp.kron(eye8, jnp.ones((16, 16), jnp.float32))
    b1t = jnp.tile(b1, 8).reshape(1, 128)
    b2t = jnp.tile(jnp.pad(b2, (0, 16 - C)), 8).reshape(1, 128)

    w1p = jnp.pad(W1, ((0, 0), (0, 128 - H)))
    y1 = _tc1(x, w1p)
    s1, g1, dinv = _make_sc_layer1(nb)(src_sl, dst_sl, y1)

    s1f = s1.reshape(2 * NF, 128)
    g1f = g1.reshape(NF, 128)
    dinvf = dinv.reshape(NF, 128)
    g2f = _tc2(s1f, g1f, dinvf, b1t, w2blk)

    s2 = _make_sc_layer2(nb)(src_sl, dst_sl, g2f.reshape(NPAD, 16))
    of = _tc3(s2.reshape(2 * NF, 128), g2f, dinvf, b2t, onesblk)
    return of.reshape(NPAD, 16)[:N, :C]


# speedup vs baseline: 1.3645x; 1.2186x over previous
import functools

import jax
import jax.numpy as jnp
from jax import lax
from jax.experimental import pallas as pl
from jax.experimental.pallas import tpu as pltpu
from jax.experimental.pallas import tpu_sc as plsc

N = 10000
D = 128
H = 16
C = 10

NC = 2
NS = 16
NW = NC * NS
BATCH = 128
NPAD = 10240
CHUNK = NPAD // NS

_MESH = plsc.VectorSubcoreMesh(core_axis_name="c", subcore_axis_name="s")
_SC_PARAMS = pltpu.CompilerParams(
    use_tc_tiling_on_sc=False, needs_layout_passes=False
)


EXTRA = 4


def _rsqrt16(v):
    i = plsc.bitcast(v, jnp.int32)
    i = jnp.int32(0x5F3759DF) - (i >> 1)
    y = plsc.bitcast(i, jnp.float32)
    for _ in range(3):
        y = y * (1.5 - 0.5 * v * y * y)
    return y


def _zero_acc_slice(zbuf, acc, s):
    def zrow(i, _):
        zbuf[i, :] = jnp.zeros((16,), jnp.float32)
        return 0

    lax.fori_loop(0, CHUNK, zrow, 0)
    pltpu.sync_copy(zbuf, acc.at[pl.ds(s * CHUNK, CHUNK)])


def _edge_pass(ev_v, g_sh, acc, bufs, gsems, ssems, nb):
    def fire_g(j, b):
        pltpu.async_copy(g_sh.at[ev_v.at[j, 0]], bufs[b], gsems[b])

    def wait_g(j, b):
        pltpu.make_async_copy(
            g_sh.at[ev_v.at[j, 0]], bufs[b], gsems[b]
        ).wait()

    def fire_s(j, b):
        pltpu.async_copy(bufs[b], acc.at[ev_v.at[j, 1]], ssems[b], add=True)

    def wait_s(j, b):
        pltpu.make_async_copy(
            bufs[b], acc.at[ev_v.at[j, 1]], ssems[b]
        ).wait()

    def step(j, b, do_wait_s, do_fire_g):
        b2 = (b + 2) % 4
        if do_wait_s:
            wait_s(j - 2, b2)
        if do_fire_g:
            fire_g(j + 2, b2)
        wait_g(j, b)
        fire_s(j, b)

    fire_g(0, 0)
    fire_g(1, 1)
    step(0, 0, False, True)
    step(1, 1, False, True)

    def body(q, _):
        j0 = 2 + q * 4
        for r in range(4):
            step(j0 + r, (2 + r) % 4, True, True)
        return 0

    nq = (nb - 6) // 4
    lax.fori_loop(0, nq, body, 0)
    for j in range(2 + 4 * nq, nb - 2):
        step(j, j % 4, True, True)
    for j in (nb - 2, nb - 1):
        step(j, j % 4, True, False)
    wait_s(nb - 2, (nb - 2) % 4)
    wait_s(nb - 1, (nb - 1) % 4)


def _make_sc_layer1(nb):
    @functools.partial(
        pl.kernel,
        out_type=(
            jax.ShapeDtypeStruct((NC, NPAD, 16), jnp.float32),
            jax.ShapeDtypeStruct((NPAD, 16), jnp.float32),
            jax.ShapeDtypeStruct((NPAD, 16), jnp.float32),
        ),
        mesh=_MESH,
        compiler_params=_SC_PARAMS,
        scratch_types=[
            pltpu.VMEM((nb + 1, 2, BATCH), jnp.int32),
            pltpu.VMEM((nb + 1, 2, BATCH), jnp.int32),
            pltpu.VMEM((BATCH,), jnp.float32),
            pltpu.VMEM((CHUNK,), jnp.float32),
            pltpu.VMEM((CHUNK,), jnp.float32),
            pltpu.VMEM((CHUNK, 16), jnp.float32),
            pltpu.VMEM((CHUNK, 16), jnp.float32),
            pltpu.VMEM((CHUNK, 16), jnp.float32),
            pltpu.VMEM((CHUNK, 16), jnp.float32),
            [pltpu.VMEM((BATCH, 16), jnp.float32) for _ in range(4)],
            pltpu.VMEM_SHARED((NPAD,), jnp.float32),
            pltpu.VMEM_SHARED((NPAD, 16), jnp.float32),
            pltpu.VMEM_SHARED((NPAD, 16), jnp.float32),
            [pltpu.SemaphoreType.DMA for _ in range(4)],
            [pltpu.SemaphoreType.DMA for _ in range(4)],
            pltpu.SemaphoreType.DMA,
        ],
    )
    def sc_layer1(
        ev_hbm, h_hbm, out_hbm, g_hbm, dinv_hbm,
        ev_v, ev2_v, ones_v, degc, dinvc, zbuf, hc, gc, dc,
        bufs, deg_sh, g_sh, acc, gsems, ssems, semc,
    ):
        c = lax.axis_index("c")
        s = lax.axis_index("s")
        wid = c * NS + s
        wid2 = (1 - c) * NS + s

        for k in range(CHUNK // 16):
            degc[pl.ds(k * 16, 16)] = jnp.zeros((16,), jnp.float32)
        for k in range(BATCH // 16):
            ones_v[pl.ds(k * 16, 16)] = jnp.ones((16,), jnp.float32)
        pltpu.sync_copy(degc, deg_sh.at[pl.ds(s * CHUNK, CHUNK)])
        _zero_acc_slice(zbuf, acc, s)
        pltpu.sync_copy(ev_hbm.at[pl.ds(wid * nb, nb)], ev_v.at[pl.ds(0, nb)])
        pltpu.sync_copy(
            ev_hbm.at[pl.ds(wid2 * nb, nb)], ev2_v.at[pl.ds(0, nb)]
        )

        @pl.when(wid < EXTRA)
        def _():
            pltpu.sync_copy(ev_hbm.at[NW * nb + wid], ev_v.at[nb])

        @pl.when(wid2 < EXTRA)
        def _():
            pltpu.sync_copy(ev_hbm.at[NW * nb + wid2], ev2_v.at[nb])

        pltpu.sync_copy(
            h_hbm.at[pl.ds(s * CHUNK, CHUNK), pl.ds(0, 16)], hc
        )
        plsc.subcore_barrier()

        def count_slab(evv, has_extra):
            def fire(j):
                pltpu.async_copy(
                    ones_v, deg_sh.at[evv.at[j, 1]], semc, add=True
                )

            def drain():
                pltpu.make_async_copy(
                    ones_v, deg_sh.at[evv.at[0, 1]], semc
                ).wait()

            for j in range(4):
                fire(j)

            def cnt(j, _):
                drain()
                fire(j)
                return 0

            lax.fori_loop(4, nb, cnt, 0)

            @pl.when(has_extra)
            def _():
                fire(nb)

            for _ in range(4):
                drain()

            @pl.when(has_extra)
            def _():
                drain()

        count_slab(ev_v, wid < EXTRA)
        count_slab(ev2_v, wid2 < EXTRA)
        plsc.subcore_barrier()

        pltpu.sync_copy(deg_sh.at[pl.ds(s * CHUNK, CHUNK)], degc)
        for k in range(CHUNK // 16):
            v = degc[pl.ds(k * 16, 16)] + 1.0
            dinvc[pl.ds(k * 16, 16)] = _rsqrt16(v)

        def brow(r, _):
            dsp = plsc.load_gather(dinvc, [jnp.full((16,), r, jnp.int32)])
            gc[r, :] = hc[r, :] * dsp
            dc[r, :] = dsp
            return 0

        lax.fori_loop(0, CHUNK, brow, 0)
        pltpu.sync_copy(gc, g_sh.at[pl.ds(s * CHUNK, CHUNK)])

        @pl.when(c == 0)
        def _():
            pltpu.sync_copy(gc, g_hbm.at[pl.ds(s * CHUNK, CHUNK)])
            pltpu.sync_copy(dc, dinv_hbm.at[pl.ds(s * CHUNK, CHUNK)])

        plsc.subcore_barrier()

        _edge_pass(ev_v, g_sh, acc, bufs, gsems, ssems, nb)

        @pl.when(wid < EXTRA)
        def _():
            pltpu.sync_copy(g_sh.at[ev_v.at[nb, 0]], bufs[0])
            pltpu.sync_copy(bufs[0], acc.at[ev_v.at[nb, 1]], add=True)

        plsc.subcore_barrier()
        pltpu.sync_copy(
            acc.at[pl.ds(s * CHUNK, CHUNK)],
            out_hbm.at[c, pl.ds(s * CHUNK, CHUNK)],
        )

    return sc_layer1


def _make_sc_layer2(nb):
    @functools.partial(
        pl.kernel,
        out_type=jax.ShapeDtypeStruct((NC, NPAD, 16), jnp.float32),
        mesh=_MESH,
        compiler_params=_SC_PARAMS,
        scratch_types=[
            pltpu.VMEM((nb + 1, 2, BATCH), jnp.int32),
            [pltpu.VMEM((BATCH, 16), jnp.float32) for _ in range(4)],
            pltpu.VMEM((CHUNK, 16), jnp.float32),
            pltpu.VMEM_SHARED((NPAD, 16), jnp.float32),
            pltpu.VMEM_SHARED((NPAD, 16), jnp.float32),
            [pltpu.SemaphoreType.DMA for _ in range(4)],
            [pltpu.SemaphoreType.DMA for _ in range(4)],
        ],
    )
    def sc_layer2(
        ev_hbm, g_hbm, out_hbm,
        ev_v, bufs, zbuf, g_sh, acc, gsems, ssems,
    ):
        c = lax.axis_index("c")
        s = lax.axis_index("s")
        wid = c * NS + s

        _zero_acc_slice(zbuf, acc, s)
        pltpu.sync_copy(ev_hbm.at[pl.ds(wid * nb, nb)], ev_v.at[pl.ds(0, nb)])

        @pl.when(wid < EXTRA)
        def _():
            pltpu.sync_copy(ev_hbm.at[NW * nb + wid], ev_v.at[nb])

        pltpu.sync_copy(
            g_hbm.at[pl.ds(s * CHUNK, CHUNK)],
            g_sh.at[pl.ds(s * CHUNK, CHUNK)],
        )
        plsc.subcore_barrier()
        _edge_pass(ev_v, g_sh, acc, bufs, gsems, ssems, nb)

        @pl.when(wid < EXTRA)
        def _():
            pltpu.sync_copy(g_sh.at[ev_v.at[nb, 0]], bufs[0])
            pltpu.sync_copy(bufs[0], acc.at[ev_v.at[nb, 1]], add=True)

        plsc.subcore_barrier()
        pltpu.sync_copy(
            acc.at[pl.ds(s * CHUNK, CHUNK)],
            out_hbm.at[c, pl.ds(s * CHUNK, CHUNK)],
        )

    return sc_layer2


NF = NPAD * 16 // 128


def _tc1_body(x_ref, w_ref, h_ref):
    h = jnp.dot(
        x_ref[...], w_ref[...],
        preferred_element_type=jnp.float32,
        precision=lax.Precision.HIGHEST,
    )
    h_ref[pl.ds(0, N), :] = h
    h_ref[pl.ds(N, NPAD - N), :] = jnp.zeros((NPAD - N, 128), jnp.float32)


def _tc2_body(s_ref, g_ref, dinv_ref, b_ref, w_ref, o_ref):
    tot = s_ref[:NF] + s_ref[NF:] + g_ref[...]
    h2 = jnp.maximum(dinv_ref[...] * tot + b_ref[...], 0.0)
    o_ref[...] = (
        jnp.dot(h2, w_ref[...], preferred_element_type=jnp.float32, precision=lax.Precision.HIGHEST)
        * dinv_ref[...]
    )


def _tc3_body(s_ref, g_ref, dinv_ref, b_ref, ones_ref, o_ref):
    o = dinv_ref[...] * (s_ref[:NF] + s_ref[NF:] + g_ref[...]) + b_ref[...]
    col = lax.broadcasted_iota(jnp.int32, o.shape, 1) % 16
    valid = col < C
    m = jnp.max(jnp.where(valid, o, -jnp.inf), axis=1, keepdims=True)
    om = o - m
    e = jnp.where(valid, jnp.exp(om), 0.0)
    ssum = jnp.dot(e, ones_ref[...], preferred_element_type=jnp.float32, precision=lax.Precision.HIGHEST)
    o_ref[...] = om - jnp.log(ssum)


def _flat_spec(rows):
    return pl.BlockSpec((rows, 128), lambda: (0, 0))


_tc1 = pl.pallas_call(
    _tc1_body,
    in_specs=[_flat_spec(N), _flat_spec(D)],
    out_specs=_flat_spec(NPAD),
    out_shape=jax.ShapeDtypeStruct((NPAD, 128), jnp.float32),
)

_tc2 = pl.pallas_call(
    _tc2_body,
    in_specs=[
        _flat_spec(2 * NF),
        _flat_spec(NF),
        _flat_spec(NF),
        pl.BlockSpec((1, 128), lambda: (0, 0)),
        _flat_spec(128),
    ],
    out_specs=_flat_spec(NF),
    out_shape=jax.ShapeDtypeStruct((NF, 128), jnp.float32),
)

_tc3 = pl.pallas_call(
    _tc3_body,
    in_specs=[
        _flat_spec(2 * NF),
        _flat_spec(NF),
        _flat_spec(NF),
        pl.BlockSpec((1, 128), lambda: (0, 0)),
        _flat_spec(128),
    ],
    out_specs=_flat_spec(NF),
    out_shape=jax.ShapeDtypeStruct((NF, 128), jnp.float32),
)


def kernel(x, edge_index, W1, b1, W2, b2):
    e = edge_index.shape[1]
    ebat = e // BATCH
    nb = ebat // NW
    eview = jnp.transpose(edge_index.reshape(2, ebat, BATCH), (1, 0, 2))

    eye8 = jnp.eye(8, dtype=jnp.float32)
    w2p = jnp.pad(W2, ((0, 0), (0, 16 - C)))
    w2blk = jnp.kron(eye8, w2p)
    onesblk = jnp.kron(eye8, jnp.ones((16, 16), jnp.float32))
    b1t = jnp.tile(b1, 8).reshape(1, 128)
    b2t = jnp.tile(jnp.pad(b2, (0, 16 - C)), 8).reshape(1, 128)

    w1p = jnp.pad(W1, ((0, 0), (0, 128 - H)))
    y1 = _tc1(x, w1p)
    s1, g1, dinv = _make_sc_layer1(nb)(eview, y1)

    s1f = s1.reshape(2 * NF, 128)
    g1f = g1.reshape(NF, 128)
    dinvf = dinv.reshape(NF, 128)
    g2f = _tc2(s1f, g1f, dinvf, b1t, w2blk)

    s2 = _make_sc_layer2(nb)(eview, g2f.reshape(NPAD, 16))
    of = _tc3(s2.reshape(2 * NF, 128), g2f, dinvf, b2t, onesblk)
    return of.reshape(NPAD, 16)[:N, :C]


# speedup vs baseline: 1.3898x; 1.0186x over previous
import functools

import jax
import jax.numpy as jnp
from jax import lax
from jax.experimental import pallas as pl
from jax.experimental.pallas import tpu as pltpu
from jax.experimental.pallas import tpu_sc as plsc

N = 10000
D = 128
H = 16
C = 10

NC = 2
NS = 16
NW = NC * NS
BATCH = 128
NPAD = 10240
CHUNK = NPAD // NS

_MESH = plsc.VectorSubcoreMesh(core_axis_name="c", subcore_axis_name="s")
_SC_PARAMS = pltpu.CompilerParams(
    use_tc_tiling_on_sc=False, needs_layout_passes=False
)


EXTRA = 4


def _rsqrt16(v):
    i = plsc.bitcast(v, jnp.int32)
    i = jnp.int32(0x5F3759DF) - (i >> 1)
    y = plsc.bitcast(i, jnp.float32)
    for _ in range(3):
        y = y * (1.5 - 0.5 * v * y * y)
    return y


def _zero_acc_slice(zbuf, acc, s):
    def zrow(i, _):
        zbuf[i, :] = jnp.zeros((16,), jnp.float32)
        return 0

    lax.fori_loop(0, CHUNK, zrow, 0)
    pltpu.sync_copy(zbuf, acc.at[pl.ds(s * CHUNK, CHUNK)])


def _edge_pass(ev_v, g_sh, acc, bufs, gsems, ssems, nb):
    def fire_g(j, b):
        pltpu.async_copy(g_sh.at[ev_v.at[j, 0]], bufs[b], gsems[b])

    def wait_g(j, b):
        pltpu.make_async_copy(
            g_sh.at[ev_v.at[j, 0]], bufs[b], gsems[b]
        ).wait()

    def fire_s(j, b):
        pltpu.async_copy(bufs[b], acc.at[ev_v.at[j, 1]], ssems[b], add=True)

    def wait_s(j, b):
        pltpu.make_async_copy(
            bufs[b], acc.at[ev_v.at[j, 1]], ssems[b]
        ).wait()

    def step(j, b, do_wait_s, do_fire_g):
        b2 = (b + 2) % 4
        if do_wait_s:
            wait_s(j - 2, b2)
        if do_fire_g:
            fire_g(j + 2, b2)
        wait_g(j, b)
        fire_s(j, b)

    fire_g(0, 0)
    fire_g(1, 1)
    step(0, 0, False, True)
    step(1, 1, False, True)

    def body(q, _):
        j0 = 2 + q * 4
        for r in range(4):
            step(j0 + r, (2 + r) % 4, True, True)
        return 0

    nq = (nb - 6) // 4
    lax.fori_loop(0, nq, body, 0)
    for j in range(2 + 4 * nq, nb - 2):
        step(j, j % 4, True, True)
    for j in (nb - 2, nb - 1):
        step(j, j % 4, True, False)
    wait_s(nb - 2, (nb - 2) % 4)
    wait_s(nb - 1, (nb - 1) % 4)


def _make_sc_layer1(nb):
    @functools.partial(
        pl.kernel,
        out_type=(
            jax.ShapeDtypeStruct((NC, NPAD, 16), jnp.float32),
            jax.ShapeDtypeStruct((NPAD, 16), jnp.float32),
            jax.ShapeDtypeStruct((NPAD, 16), jnp.float32),
        ),
        mesh=_MESH,
        compiler_params=_SC_PARAMS,
        scratch_types=[
            pltpu.VMEM((nb + 1, 2, BATCH), jnp.int32),
            pltpu.VMEM((nb + 1, 2, BATCH), jnp.int32),
            pltpu.VMEM((BATCH,), jnp.float32),
            pltpu.VMEM((CHUNK,), jnp.float32),
            pltpu.VMEM((CHUNK,), jnp.float32),
            pltpu.VMEM((CHUNK, 16), jnp.float32),
            pltpu.VMEM((CHUNK, 16), jnp.float32),
            pltpu.VMEM((CHUNK, 16), jnp.float32),
            pltpu.VMEM((CHUNK, 16), jnp.float32),
            [pltpu.VMEM((BATCH, 16), jnp.float32) for _ in range(4)],
            pltpu.VMEM_SHARED((NPAD,), jnp.float32),
            pltpu.VMEM_SHARED((NPAD, 16), jnp.float32),
            pltpu.VMEM_SHARED((NPAD, 16), jnp.float32),
            [pltpu.SemaphoreType.DMA for _ in range(4)],
            [pltpu.SemaphoreType.DMA for _ in range(4)],
            pltpu.SemaphoreType.DMA,
        ],
    )
    def sc_layer1(
        ev_hbm, h_hbm, out_hbm, g_hbm, dinv_hbm,
        ev_v, ev2_v, ones_v, degc, dinvc, zbuf, hc, gc, dc,
        bufs, deg_sh, g_sh, acc, gsems, ssems, semc,
    ):
        c = lax.axis_index("c")
        s = lax.axis_index("s")
        wid = c * NS + s
        wid2 = (1 - c) * NS + s

        for k in range(CHUNK // 16):
            degc[pl.ds(k * 16, 16)] = jnp.zeros((16,), jnp.float32)
        for k in range(BATCH // 16):
            ones_v[pl.ds(k * 16, 16)] = jnp.ones((16,), jnp.float32)
        pltpu.sync_copy(degc, deg_sh.at[pl.ds(s * CHUNK, CHUNK)])
        _zero_acc_slice(zbuf, acc, s)
        pltpu.sync_copy(ev_hbm.at[pl.ds(wid * nb, nb)], ev_v.at[pl.ds(0, nb)])
        pltpu.sync_copy(
            ev_hbm.at[pl.ds(wid2 * nb, nb)], ev2_v.at[pl.ds(0, nb)]
        )

        @pl.when(wid < EXTRA)
        def _():
            pltpu.sync_copy(ev_hbm.at[NW * nb + wid], ev_v.at[nb])

        @pl.when(wid2 < EXTRA)
        def _():
            pltpu.sync_copy(ev_hbm.at[NW * nb + wid2], ev2_v.at[nb])

        pltpu.sync_copy(
            h_hbm.at[pl.ds(s * CHUNK, CHUNK), pl.ds(0, 16)], hc
        )
        plsc.subcore_barrier()

        def count_slab(evv, has_extra):
            def fire(j):
                pltpu.async_copy(
                    ones_v, deg_sh.at[evv.at[j, 1]], semc, add=True
                )

            def drain():
                pltpu.make_async_copy(
                    ones_v, deg_sh.at[evv.at[0, 1]], semc
                ).wait()

            for j in range(8):
                fire(j)

            def cnt(j, _):
                drain()
                fire(j)
                return 0

            lax.fori_loop(8, nb, cnt, 0)

            @pl.when(has_extra)
            def _():
                fire(nb)

            for _ in range(8):
                drain()

            @pl.when(has_extra)
            def _():
                drain()

        count_slab(ev_v, wid < EXTRA)
        count_slab(ev2_v, wid2 < EXTRA)
        plsc.subcore_barrier()

        pltpu.sync_copy(deg_sh.at[pl.ds(s * CHUNK, CHUNK)], degc)
        for k in range(CHUNK // 16):
            v = degc[pl.ds(k * 16, 16)] + 1.0
            dinvc[pl.ds(k * 16, 16)] = _rsqrt16(v)

        def brow(q, _):
            r0 = q * 4
            for k in range(4):
                r = r0 + k
                dsp = plsc.load_gather(
                    dinvc, [jnp.full((16,), r, jnp.int32)]
                )
                gc[r, :] = hc[r, :] * dsp
                dc[r, :] = dsp
            return 0

        lax.fori_loop(0, CHUNK // 4, brow, 0)
        pltpu.sync_copy(gc, g_sh.at[pl.ds(s * CHUNK, CHUNK)])

        @pl.when(c == 0)
        def _():
            pltpu.sync_copy(gc, g_hbm.at[pl.ds(s * CHUNK, CHUNK)])
            pltpu.sync_copy(dc, dinv_hbm.at[pl.ds(s * CHUNK, CHUNK)])

        plsc.subcore_barrier()

        _edge_pass(ev_v, g_sh, acc, bufs, gsems, ssems, nb)

        @pl.when(wid < EXTRA)
        def _():
            pltpu.sync_copy(g_sh.at[ev_v.at[nb, 0]], bufs[0])
            pltpu.sync_copy(bufs[0], acc.at[ev_v.at[nb, 1]], add=True)

        plsc.subcore_barrier()
        pltpu.sync_copy(
            acc.at[pl.ds(s * CHUNK, CHUNK)],
            out_hbm.at[c, pl.ds(s * CHUNK, CHUNK)],
        )

    return sc_layer1


def _make_sc_layer2(nb):
    @functools.partial(
        pl.kernel,
        out_type=jax.ShapeDtypeStruct((NC, NPAD, 16), jnp.float32),
        mesh=_MESH,
        compiler_params=_SC_PARAMS,
        scratch_types=[
            pltpu.VMEM((nb + 1, 2, BATCH), jnp.int32),
            [pltpu.VMEM((BATCH, 16), jnp.float32) for _ in range(4)],
            pltpu.VMEM((CHUNK, 16), jnp.float32),
            pltpu.VMEM_SHARED((NPAD, 16), jnp.float32),
            pltpu.VMEM_SHARED((NPAD, 16), jnp.float32),
            [pltpu.SemaphoreType.DMA for _ in range(4)],
            [pltpu.SemaphoreType.DMA for _ in range(4)],
        ],
    )
    def sc_layer2(
        ev_hbm, g_hbm, out_hbm,
        ev_v, bufs, zbuf, g_sh, acc, gsems, ssems,
    ):
        c = lax.axis_index("c")
        s = lax.axis_index("s")
        wid = c * NS + s

        _zero_acc_slice(zbuf, acc, s)
        pltpu.sync_copy(ev_hbm.at[pl.ds(wid * nb, nb)], ev_v.at[pl.ds(0, nb)])

        @pl.when(wid < EXTRA)
        def _():
            pltpu.sync_copy(ev_hbm.at[NW * nb + wid], ev_v.at[nb])

        pltpu.sync_copy(
            g_hbm.at[pl.ds(s * CHUNK, CHUNK)],
            g_sh.at[pl.ds(s * CHUNK, CHUNK)],
        )
        plsc.subcore_barrier()
        _edge_pass(ev_v, g_sh, acc, bufs, gsems, ssems, nb)

        @pl.when(wid < EXTRA)
        def _():
            pltpu.sync_copy(g_sh.at[ev_v.at[nb, 0]], bufs[0])
            pltpu.sync_copy(bufs[0], acc.at[ev_v.at[nb, 1]], add=True)

        plsc.subcore_barrier()
        pltpu.sync_copy(
            acc.at[pl.ds(s * CHUNK, CHUNK)],
            out_hbm.at[c, pl.ds(s * CHUNK, CHUNK)],
        )

    return sc_layer2


NF = NPAD * 16 // 128


def _tc1_body(x_ref, w_ref, h_ref):
    h = jnp.dot(
        x_ref[...], w_ref[...],
        preferred_element_type=jnp.float32,
        precision=lax.Precision.HIGHEST,
    )
    h_ref[pl.ds(0, N), :] = h
    h_ref[pl.ds(N, NPAD - N), :] = jnp.zeros((NPAD - N, 128), jnp.float32)


def _tc2_body(s_ref, g_ref, dinv_ref, b_ref, w_ref, o_ref):
    tot = s_ref[:NF] + s_ref[NF:] + g_ref[...]
    h2 = jnp.maximum(dinv_ref[...] * tot + b_ref[...], 0.0)
    o_ref[...] = (
        jnp.dot(h2, w_ref[...], preferred_element_type=jnp.float32, precision=lax.Precision.HIGHEST)
        * dinv_ref[...]
    )


def _tc3_body(s_ref, g_ref, dinv_ref, b_ref, ones_ref, o_ref):
    o = dinv_ref[...] * (s_ref[:NF] + s_ref[NF:] + g_ref[...]) + b_ref[...]
    col = lax.broadcasted_iota(jnp.int32, o.shape, 1) % 16
    valid = col < C
    m = jnp.max(jnp.where(valid, o, -jnp.inf), axis=1, keepdims=True)
    om = o - m
    e = jnp.where(valid, jnp.exp(om), 0.0)
    ssum = jnp.dot(e, ones_ref[...], preferred_element_type=jnp.float32, precision=lax.Precision.HIGHEST)
    o_ref[...] = om - jnp.log(ssum)


def _flat_spec(rows):
    return pl.BlockSpec((rows, 128), lambda: (0, 0))


_tc1 = pl.pallas_call(
    _tc1_body,
    in_specs=[_flat_spec(N), _flat_spec(D)],
    out_specs=_flat_spec(NPAD),
    out_shape=jax.ShapeDtypeStruct((NPAD, 128), jnp.float32),
)

_tc2 = pl.pallas_call(
    _tc2_body,
    in_specs=[
        _flat_spec(2 * NF),
        _flat_spec(NF),
        _flat_spec(NF),
        pl.BlockSpec((1, 128), lambda: (0, 0)),
        _flat_spec(128),
    ],
    out_specs=_flat_spec(NF),
    out_shape=jax.ShapeDtypeStruct((NF, 128), jnp.float32),
)

_tc3 = pl.pallas_call(
    _tc3_body,
    in_specs=[
        _flat_spec(2 * NF),
        _flat_spec(NF),
        _flat_spec(NF),
        pl.BlockSpec((1, 128), lambda: (0, 0)),
        _flat_spec(128),
    ],
    out_specs=_flat_spec(NF),
    out_shape=jax.ShapeDtypeStruct((NF, 128), jnp.float32),
)


def kernel(x, edge_index, W1, b1, W2, b2):
    e = edge_index.shape[1]
    ebat = e // BATCH
    nb = ebat // NW
    eview = jnp.transpose(edge_index.reshape(2, ebat, BATCH), (1, 0, 2))

    eye8 = jnp.eye(8, dtype=jnp.float32)
    w2p = jnp.pad(W2, ((0, 0), (0, 16 - C)))
    w2blk = jnp.kron(eye8, w2p)
    onesblk = jnp.kron(eye8, jnp.ones((16, 16), jnp.float32))
    b1t = jnp.tile(b1, 8).reshape(1, 128)
    b2t = jnp.tile(jnp.pad(b2, (0, 16 - C)), 8).reshape(1, 128)

    w1p = jnp.pad(W1, ((0, 0), (0, 128 - H)))
    y1 = _tc1(x, w1p)
    s1, g1, dinv = _make_sc_layer1(nb)(eview, y1)

    s1f = s1.reshape(2 * NF, 128)
    g1f = g1.reshape(NF, 128)
    dinvf = dinv.reshape(NF, 128)
    g2f = _tc2(s1f, g1f, dinvf, b1t, w2blk)

    s2 = _make_sc_layer2(nb)(eview, g2f.reshape(NPAD, 16))
    of = _tc3(s2.reshape(2 * NF, 128), g2f, dinvf, b2t, onesblk)
    return of.reshape(NPAD, 16)[:N, :C]
